# baseline (speedup 1.0000x reference)
import jax
import jax.numpy as jnp
from jax import lax
from jax.experimental import pallas as pl
from jax.experimental.pallas import tpu as pltpu
from jax.experimental.pallas import tpu_sc as plsc

N_C = 4096
N_F = 16384
D = 64
NB = 8
NW = 32
MPT = N_F // NW
GPT = MPT // 16
CHUNK = 128
BLK = 1024


def _splat(ref, idx_scalar):
    return plsc.load_gather(ref, [jnp.full((16,), idx_scalar, jnp.int32)])


def _knn_body(cx_h, cy_h, cz_h, cb_h, sx_h, sy_h, sz_h, fb_h, segs_h, x_h,
              f1_h, f2_h, f3_h, nw_h,
              cx_v, cy_v, cz_v, cb_v, sx_v, sy_v, sz_v, fb_v, segs_v,
              i1_v, i2_v, i3_v, w1_v, w2_v, w3_v, r1_v, r2_v, r3_v, sem):
    wid = lax.axis_index("s") * 2 + lax.axis_index("c")
    base = wid * MPT

    pltpu.sync_copy(cx_h, cx_v)
    pltpu.sync_copy(cy_h, cy_v)
    pltpu.sync_copy(cz_h, cz_v)
    pltpu.sync_copy(cb_h, cb_v)
    pltpu.sync_copy(sx_h.at[pl.ds(base, MPT)], sx_v)
    pltpu.sync_copy(sy_h.at[pl.ds(base, MPT)], sy_v)
    pltpu.sync_copy(sz_h.at[pl.ds(base, MPT)], sz_v)
    pltpu.sync_copy(fb_h.at[pl.ds(base, MPT)], fb_v)
    pltpu.sync_copy(segs_h, segs_v)

    def group_body(g, _):
        o = g * 16
        px = sx_v[pl.ds(o, 16)]
        py = sy_v[pl.ds(o, 16)]
        pz = sz_v[pl.ds(o, 16)]
        vb = fb_v[pl.ds(o, 16)]
        bmin = vb[0]
        bmax = vb[15]
        s = _splat(segs_v, bmin)[0]
        e = _splat(segs_v, bmax + 1)[0]

        big = jnp.full((16,), 1e30, jnp.float32)
        zero = jnp.zeros((16,), jnp.int32)
        init6 = (big, big, big, zero, zero, zero)

        def dist(jv):
            dx = px - plsc.load_gather(cx_v, [jv])
            dy = py - plsc.load_gather(cy_v, [jv])
            dz = pz - plsc.load_gather(cz_v, [jv])
            return dx * dx + dy * dy + dz * dz

        def insert(d, jv, acc):
            d1, d2, d3, i1, i2, i3 = acc
            c1 = d < d1
            c2 = d < d2
            c3 = d < d3
            return (jnp.where(c1, d, d1),
                    jnp.where(c1, d1, jnp.where(c2, d, d2)),
                    jnp.where(c2, d2, jnp.where(c3, d, d3)),
                    jnp.where(c1, jv, i1),
                    jnp.where(c1, i1, jnp.where(c2, jv, i2)),
                    jnp.where(c2, i2, jnp.where(c3, jv, i3)))

        def merge_one(d, jv, acc):
            d1, d2, d3, i1, i2, i3 = acc
            c1 = (d < d1) | ((d == d1) & (jv < i1))
            c2 = (d < d2) | ((d == d2) & (jv < i2))
            c3 = (d < d3) | ((d == d3) & (jv < i3))
            return (jnp.where(c1, d, d1),
                    jnp.where(c1, d1, jnp.where(c2, d, d2)),
                    jnp.where(c2, d2, jnp.where(c3, d, d3)),
                    jnp.where(c1, jv, i1),
                    jnp.where(c1, i1, jnp.where(c2, jv, i2)),
                    jnp.where(c2, i2, jnp.where(c3, jv, i3)))

        def fast_scan():
            def chunk_body(k, carry):
                off = s + k * 16
                cxc = cx_v[pl.ds(off, 16)]
                cyc = cy_v[pl.ds(off, 16)]
                czc = cz_v[pl.ds(off, 16)]
                a = carry[0:6]
                b = carry[6:12]
                ja = carry[12]
                jb = carry[13]
                for u in range(16):
                    dxs = px - jnp.full((16,), cxc[u], jnp.float32)
                    dys = py - jnp.full((16,), cyc[u], jnp.float32)
                    dzs = pz - jnp.full((16,), czc[u], jnp.float32)
                    d = dxs * dxs + dys * dys + dzs * dzs
                    if u % 2 == 0:
                        a = insert(d, ja, a)
                        ja = ja + 2
                    else:
                        b = insert(d, jb, b)
                        jb = jb + 2
                return a + b + (ja, jb)
            h = (e - s) // 16
            js = jnp.full((16,), s, jnp.int32)
            acc = lax.fori_loop(0, h, chunk_body, init6 + init6 + (js, js + 1))

            def rem_body(j, carry):
                return insert(dist(carry[6]), carry[6], carry[0:6]) + (carry[6] + 1,)
            acc_a = lax.fori_loop(s + 16 * h, e, rem_body, acc[0:6] + (acc[12],))
            a = acc_a[0:6]
            b = acc[6:12]
            a = merge_one(b[0], b[3], a)
            a = merge_one(b[1], b[4], a)
            a = merge_one(b[2], b[5], a)
            return a

        def slow_scan():
            def cand_body(j, carry):
                jv = carry[6]
                d = dist(jv)
                d = jnp.where(vb != plsc.load_gather(cb_v, [jv]),
                              jnp.float32(1e10), d)
                return insert(d, jv, carry[0:6]) + (jv + 1,)
            js = jnp.full((16,), s, jnp.int32)
            return lax.fori_loop(s, e, cand_body, init6 + (js,))[0:6]

        d1, d2, d3, i1, i2, i3 = lax.cond(bmin == bmax, fast_scan, slow_scan)

        w1 = 1.0 / jnp.maximum(d1, 1e-16)
        w2 = 1.0 / jnp.maximum(d2, 1e-16)
        w3 = 1.0 / jnp.maximum(d3, 1e-16)
        inv = 1.0 / (w1 + w2 + w3)
        w1_v[pl.ds(o, 16)] = w1 * inv
        w2_v[pl.ds(o, 16)] = w2 * inv
        w3_v[pl.ds(o, 16)] = w3 * inv
        i1_v[pl.ds(o, 16)] = i1
        i2_v[pl.ds(o, 16)] = i2
        i3_v[pl.ds(o, 16)] = i3
        return 0

    lax.fori_loop(0, GPT, group_body, 0)

    pltpu.sync_copy(w1_v, nw_h.at[0, pl.ds(base, MPT)])
    pltpu.sync_copy(w2_v, nw_h.at[1, pl.ds(base, MPT)])
    pltpu.sync_copy(w3_v, nw_h.at[2, pl.ds(base, MPT)])

    for c in range(MPT // CHUNK):
        off = c * CHUNK
        cp1 = pltpu.async_copy(x_h.at[i1_v.at[pl.ds(off, CHUNK)]], r1_v, sem)
        cp2 = pltpu.async_copy(x_h.at[i2_v.at[pl.ds(off, CHUNK)]], r2_v, sem)
        cp3 = pltpu.async_copy(x_h.at[i3_v.at[pl.ds(off, CHUNK)]], r3_v, sem)
        cp1.wait()
        cp2.wait()
        cp3.wait()
        pltpu.sync_copy(r1_v, f1_h.at[pl.ds(base + off, CHUNK)])
        pltpu.sync_copy(r2_v, f2_h.at[pl.ds(base + off, CHUNK)])
        pltpu.sync_copy(r3_v, f3_h.at[pl.ds(base + off, CHUNK)])


_knn_call = pl.kernel(
    _knn_body,
    out_type=(
        jax.ShapeDtypeStruct((N_F, D), jnp.float32),
        jax.ShapeDtypeStruct((N_F, D), jnp.float32),
        jax.ShapeDtypeStruct((N_F, D), jnp.float32),
        jax.ShapeDtypeStruct((3, N_F), jnp.float32),
    ),
    mesh=plsc.VectorSubcoreMesh(core_axis_name="c", subcore_axis_name="s",
                                num_cores=2, num_subcores=16),
    compiler_params=pltpu.CompilerParams(needs_layout_passes=False,
                                         use_tc_tiling_on_sc=False),
    scratch_types=[
        pltpu.VMEM((N_C,), jnp.float32),
        pltpu.VMEM((N_C,), jnp.float32),
        pltpu.VMEM((N_C,), jnp.float32),
        pltpu.VMEM((N_C,), jnp.int32),
        pltpu.VMEM((MPT,), jnp.float32),
        pltpu.VMEM((MPT,), jnp.float32),
        pltpu.VMEM((MPT,), jnp.float32),
        pltpu.VMEM((MPT,), jnp.int32),
        pltpu.VMEM((16,), jnp.int32),
        pltpu.VMEM((MPT,), jnp.int32),
        pltpu.VMEM((MPT,), jnp.int32),
        pltpu.VMEM((MPT,), jnp.int32),
        pltpu.VMEM((MPT,), jnp.float32),
        pltpu.VMEM((MPT,), jnp.float32),
        pltpu.VMEM((MPT,), jnp.float32),
        pltpu.VMEM((CHUNK, D), jnp.float32),
        pltpu.VMEM((CHUNK, D), jnp.float32),
        pltpu.VMEM((CHUNK, D), jnp.float32),
        pltpu.SemaphoreType.DMA,
    ],
)


def _mlp_body(f1_ref, f2_ref, f3_ref, nw_ref, xs_ref, w1_ref, b1_ref, out_ref):
    nw1 = nw_ref[0, :][:, None]
    nw2 = nw_ref[1, :][:, None]
    nw3 = nw_ref[2, :][:, None]
    y = nw1 * f1_ref[...] + nw2 * f2_ref[...] + nw3 * f3_ref[...]
    h = jax.lax.dot(y, w1_ref[0:D, :], preferred_element_type=jnp.float32)
    h = h + jax.lax.dot(xs_ref[...], w1_ref[D:, :], preferred_element_type=jnp.float32)
    out_ref[...] = jnp.maximum(h + b1_ref[0, :][None, :], 0.0)


def kernel(x, pos, batch, seed_idx, x_skip, pos_skip, batch_skip, seed_idx_skip, W1, b1):
    pos = pos.astype(jnp.float32)
    ps = pos_skip.astype(jnp.float32)
    bi = batch.astype(jnp.int32)
    fbi = batch_skip.astype(jnp.int32)
    segs = jnp.sum(bi[None, :] < jnp.arange(16, dtype=jnp.int32)[:, None],
                   axis=1).astype(jnp.int32)

    f1, f2, f3, nw = _knn_call(
        pos[:, 0], pos[:, 1], pos[:, 2], bi,
        ps[:, 0], ps[:, 1], ps[:, 2], fbi,
        segs, x.astype(jnp.float32))

    b1r = b1.reshape(1, -1)

    out = pl.pallas_call(
        _mlp_body,
        grid=(N_F // BLK,),
        in_specs=[
            pl.BlockSpec((BLK, D), lambda i: (i, 0)),
            pl.BlockSpec((BLK, D), lambda i: (i, 0)),
            pl.BlockSpec((BLK, D), lambda i: (i, 0)),
            pl.BlockSpec((3, BLK), lambda i: (0, i)),
            pl.BlockSpec((BLK, D), lambda i: (i, 0)),
            pl.BlockSpec((2 * D, 2 * D), lambda i: (0, 0)),
            pl.BlockSpec((1, 2 * D), lambda i: (0, 0)),
        ],
        out_specs=pl.BlockSpec((BLK, 2 * D), lambda i: (i, 0)),
        out_shape=jax.ShapeDtypeStruct((N_F, 2 * D), jnp.float32),
    )(f1, f2, f3, nw, x_skip, W1, b1r)
    return (out, pos_skip, batch_skip)


# speedup vs baseline: 1.1384x; 1.1384x over previous
import jax
import jax.numpy as jnp
from jax import lax
from jax.experimental import pallas as pl
from jax.experimental.pallas import tpu as pltpu
from jax.experimental.pallas import tpu_sc as plsc

N_C = 4096
N_F = 16384
D = 64
DP = 128
NB = 8
NW = 32
MPT = N_F // NW
GPT = MPT // 16
CHUNK = 128
BLK = 1024


def _splat(ref, idx_scalar):
    return plsc.load_gather(ref, [jnp.full((16,), idx_scalar, jnp.int32)])


def _knn_body(cx_h, cy_h, cz_h, cb_h, sx_h, sy_h, sz_h, fb_h, segs_h, x_h,
              f1_h, f2_h, f3_h, nw1_h, nw2_h, nw3_h,
              cx_v, cy_v, cz_v, cb_v, sx_v, sy_v, sz_v, fb_v, segs_v,
              i1_v, i2_v, i3_v, w1_v, w2_v, w3_v, r1_v, r2_v, r3_v, sem):
    wid = lax.axis_index("s") * 2 + lax.axis_index("c")
    base = wid * MPT

    pltpu.sync_copy(cx_h, cx_v)
    pltpu.sync_copy(cy_h, cy_v)
    pltpu.sync_copy(cz_h, cz_v)
    pltpu.sync_copy(cb_h, cb_v)
    pltpu.sync_copy(sx_h.at[pl.ds(base, MPT)], sx_v)
    pltpu.sync_copy(sy_h.at[pl.ds(base, MPT)], sy_v)
    pltpu.sync_copy(sz_h.at[pl.ds(base, MPT)], sz_v)
    pltpu.sync_copy(fb_h.at[pl.ds(base, MPT)], fb_v)
    pltpu.sync_copy(segs_h, segs_v)

    def group_body(g, _):
        o = g * 16
        px = sx_v[pl.ds(o, 16)]
        py = sy_v[pl.ds(o, 16)]
        pz = sz_v[pl.ds(o, 16)]
        vb = fb_v[pl.ds(o, 16)]
        bmin = vb[0]
        bmax = vb[15]
        s = _splat(segs_v, bmin)[0]
        e = _splat(segs_v, bmax + 1)[0]

        big = jnp.full((16,), 1e30, jnp.float32)
        zero = jnp.zeros((16,), jnp.int32)
        init6 = (big, big, big, zero, zero, zero)

        def dist(jv):
            dx = px - plsc.load_gather(cx_v, [jv])
            dy = py - plsc.load_gather(cy_v, [jv])
            dz = pz - plsc.load_gather(cz_v, [jv])
            return dx * dx + dy * dy + dz * dz

        def insert(d, jv, acc):
            d1, d2, d3, i1, i2, i3 = acc
            c1 = d < d1
            c2 = d < d2
            c3 = d < d3
            return (jnp.where(c1, d, d1),
                    jnp.where(c1, d1, jnp.where(c2, d, d2)),
                    jnp.where(c2, d2, jnp.where(c3, d, d3)),
                    jnp.where(c1, jv, i1),
                    jnp.where(c1, i1, jnp.where(c2, jv, i2)),
                    jnp.where(c2, i2, jnp.where(c3, jv, i3)))

        def merge_one(d, jv, acc):
            d1, d2, d3, i1, i2, i3 = acc
            c1 = (d < d1) | ((d == d1) & (jv < i1))
            c2 = (d < d2) | ((d == d2) & (jv < i2))
            c3 = (d < d3) | ((d == d3) & (jv < i3))
            return (jnp.where(c1, d, d1),
                    jnp.where(c1, d1, jnp.where(c2, d, d2)),
                    jnp.where(c2, d2, jnp.where(c3, d, d3)),
                    jnp.where(c1, jv, i1),
                    jnp.where(c1, i1, jnp.where(c2, jv, i2)),
                    jnp.where(c2, i2, jnp.where(c3, jv, i3)))

        def fast_scan():
            def quad_body(i, carry):
                a = carry[0:6]
                b = carry[6:12]
                ja = carry[12]
                jb = carry[13]
                a = insert(dist(ja), ja, a)
                b = insert(dist(jb), jb, b)
                ja = ja + 2
                jb = jb + 2
                a = insert(dist(ja), ja, a)
                b = insert(dist(jb), jb, b)
                return a + b + (ja + 2, jb + 2)
            h = (e - s) // 4
            js = jnp.full((16,), s, jnp.int32)
            acc = lax.fori_loop(0, h, quad_body, init6 + init6 + (js, js + 1))

            def rem_body(j, carry):
                return insert(dist(carry[6]), carry[6], carry[0:6]) + (carry[6] + 1,)
            acc_a = lax.fori_loop(s + 4 * h, e, rem_body, acc[0:6] + (acc[12],))
            a = acc_a[0:6]
            b = acc[6:12]
            a = merge_one(b[0], b[3], a)
            a = merge_one(b[1], b[4], a)
            a = merge_one(b[2], b[5], a)
            return a

        def slow_scan():
            def cand_body(j, carry):
                jv = carry[6]
                d = dist(jv)
                d = jnp.where(vb != plsc.load_gather(cb_v, [jv]),
                              jnp.float32(1e10), d)
                return insert(d, jv, carry[0:6]) + (jv + 1,)
            js = jnp.full((16,), s, jnp.int32)
            return lax.fori_loop(s, e, cand_body, init6 + (js,))[0:6]

        d1, d2, d3, i1, i2, i3 = lax.cond(bmin == bmax, fast_scan, slow_scan)

        w1 = 1.0 / jnp.maximum(d1, 1e-16)
        w2 = 1.0 / jnp.maximum(d2, 1e-16)
        w3 = 1.0 / jnp.maximum(d3, 1e-16)
        inv = 1.0 / (w1 + w2 + w3)
        w1_v[pl.ds(o, 16)] = w1 * inv
        w2_v[pl.ds(o, 16)] = w2 * inv
        w3_v[pl.ds(o, 16)] = w3 * inv
        i1_v[pl.ds(o, 16)] = i1
        i2_v[pl.ds(o, 16)] = i2
        i3_v[pl.ds(o, 16)] = i3
        return 0

    lax.fori_loop(0, GPT, group_body, 0)

    pltpu.sync_copy(w1_v, nw1_h.at[pl.ds(base, MPT)])
    pltpu.sync_copy(w2_v, nw2_h.at[pl.ds(base, MPT)])
    pltpu.sync_copy(w3_v, nw3_h.at[pl.ds(base, MPT)])

    for c in range(MPT // CHUNK):
        off = c * CHUNK
        cp1 = pltpu.async_copy(x_h.at[i1_v.at[pl.ds(off, CHUNK)]], r1_v, sem)
        cp2 = pltpu.async_copy(x_h.at[i2_v.at[pl.ds(off, CHUNK)]], r2_v, sem)
        cp3 = pltpu.async_copy(x_h.at[i3_v.at[pl.ds(off, CHUNK)]], r3_v, sem)
        cp1.wait()
        cp2.wait()
        cp3.wait()
        pltpu.sync_copy(r1_v, f1_h.at[pl.ds(base + off, CHUNK)])
        pltpu.sync_copy(r2_v, f2_h.at[pl.ds(base + off, CHUNK)])
        pltpu.sync_copy(r3_v, f3_h.at[pl.ds(base + off, CHUNK)])


_knn_call = pl.kernel(
    _knn_body,
    out_type=(
        jax.ShapeDtypeStruct((N_F, DP), jnp.float32),
        jax.ShapeDtypeStruct((N_F, DP), jnp.float32),
        jax.ShapeDtypeStruct((N_F, DP), jnp.float32),
        jax.ShapeDtypeStruct((N_F,), jnp.float32),
        jax.ShapeDtypeStruct((N_F,), jnp.float32),
        jax.ShapeDtypeStruct((N_F,), jnp.float32),
    ),
    mesh=plsc.VectorSubcoreMesh(core_axis_name="c", subcore_axis_name="s",
                                num_cores=2, num_subcores=16),
    compiler_params=pltpu.CompilerParams(needs_layout_passes=False,
                                         use_tc_tiling_on_sc=True),
    scratch_types=[
        pltpu.VMEM((N_C,), jnp.float32),
        pltpu.VMEM((N_C,), jnp.float32),
        pltpu.VMEM((N_C,), jnp.float32),
        pltpu.VMEM((N_C,), jnp.int32),
        pltpu.VMEM((MPT,), jnp.float32),
        pltpu.VMEM((MPT,), jnp.float32),
        pltpu.VMEM((MPT,), jnp.float32),
        pltpu.VMEM((MPT,), jnp.int32),
        pltpu.VMEM((16,), jnp.int32),
        pltpu.VMEM((MPT,), jnp.int32),
        pltpu.VMEM((MPT,), jnp.int32),
        pltpu.VMEM((MPT,), jnp.int32),
        pltpu.VMEM((MPT,), jnp.float32),
        pltpu.VMEM((MPT,), jnp.float32),
        pltpu.VMEM((MPT,), jnp.float32),
        pltpu.VMEM((CHUNK, DP), jnp.float32),
        pltpu.VMEM((CHUNK, DP), jnp.float32),
        pltpu.VMEM((CHUNK, DP), jnp.float32),
        pltpu.SemaphoreType.DMA,
    ],
)


def _mlp_body(f1_ref, f2_ref, f3_ref, nw_ref, xs_ref, w1_ref, b1_ref, out_ref):
    nw1 = nw_ref[0, :][:, None]
    nw2 = nw_ref[1, :][:, None]
    nw3 = nw_ref[2, :][:, None]
    y = (nw1 * f1_ref[:, 0:D] + nw2 * f2_ref[:, 0:D] + nw3 * f3_ref[:, 0:D])
    h = jax.lax.dot(y, w1_ref[0:D, :], preferred_element_type=jnp.float32)
    h = h + jax.lax.dot(xs_ref[...], w1_ref[D:, :], preferred_element_type=jnp.float32)
    out_ref[...] = jnp.maximum(h + b1_ref[0, :][None, :], 0.0)


def kernel(x, pos, batch, seed_idx, x_skip, pos_skip, batch_skip, seed_idx_skip, W1, b1):
    pos = pos.astype(jnp.float32)
    ps = pos_skip.astype(jnp.float32)
    bi = batch.astype(jnp.int32)
    fbi = batch_skip.astype(jnp.int32)
    segs = jnp.sum(bi[None, :] < jnp.arange(16, dtype=jnp.int32)[:, None],
                   axis=1).astype(jnp.int32)
    xp = jnp.pad(x.astype(jnp.float32), ((0, 0), (0, DP - D)))

    f1, f2, f3, nw1, nw2, nw3 = _knn_call(
        pos[:, 0], pos[:, 1], pos[:, 2], bi,
        ps[:, 0], ps[:, 1], ps[:, 2], fbi,
        segs, xp)

    nw = jnp.stack([nw1, nw2, nw3])
    b1r = b1.reshape(1, -1)

    out = pl.pallas_call(
        _mlp_body,
        grid=(N_F // BLK,),
        in_specs=[
            pl.BlockSpec((BLK, DP), lambda i: (i, 0)),
            pl.BlockSpec((BLK, DP), lambda i: (i, 0)),
            pl.BlockSpec((BLK, DP), lambda i: (i, 0)),
            pl.BlockSpec((3, BLK), lambda i: (0, i)),
            pl.BlockSpec((BLK, D), lambda i: (i, 0)),
            pl.BlockSpec((2 * D, 2 * D), lambda i: (0, 0)),
            pl.BlockSpec((1, 2 * D), lambda i: (0, 0)),
        ],
        out_specs=pl.BlockSpec((BLK, 2 * D), lambda i: (i, 0)),
        out_shape=jax.ShapeDtypeStruct((N_F, 2 * D), jnp.float32),
    )(f1, f2, f3, nw, x_skip, W1, b1r)
    return (out, pos_skip, batch_skip)


# speedup vs baseline: 1.2010x; 1.0550x over previous
import jax
import jax.numpy as jnp
from jax import lax
from jax.experimental import pallas as pl
from jax.experimental.pallas import tpu as pltpu
from jax.experimental.pallas import tpu_sc as plsc

N_C = 4096
N_F = 16384
D = 64
DP = 128
NB = 8
NW = 32
MPT = N_F // NW
GPT = MPT // 16
CHUNK = 128
BLK = 1024


def _splat(ref, idx_scalar):
    return plsc.load_gather(ref, [jnp.full((16,), idx_scalar, jnp.int32)])


def _knn_body(cx_h, cy_h, cz_h, cb_h, sx_h, sy_h, sz_h, fb_h, segs_h, x_h,
              y_h,
              cx_v, cy_v, cz_v, cb_v, sx_v, sy_v, sz_v, fb_v, segs_v,
              i1_v, i2_v, i3_v, w1_v, w2_v, w3_v, r1_v, r2_v, r3_v, y_v, sem):
    wid = lax.axis_index("s") * 2 + lax.axis_index("c")
    base = wid * MPT

    pltpu.sync_copy(cx_h, cx_v)
    pltpu.sync_copy(cy_h, cy_v)
    pltpu.sync_copy(cz_h, cz_v)
    pltpu.sync_copy(cb_h, cb_v)
    pltpu.sync_copy(sx_h.at[pl.ds(base, MPT)], sx_v)
    pltpu.sync_copy(sy_h.at[pl.ds(base, MPT)], sy_v)
    pltpu.sync_copy(sz_h.at[pl.ds(base, MPT)], sz_v)
    pltpu.sync_copy(fb_h.at[pl.ds(base, MPT)], fb_v)
    pltpu.sync_copy(segs_h, segs_v)

    def group_body(g, _):
        o = g * 16
        px = sx_v[pl.ds(o, 16)]
        py = sy_v[pl.ds(o, 16)]
        pz = sz_v[pl.ds(o, 16)]
        vb = fb_v[pl.ds(o, 16)]
        bmin = vb[0]
        bmax = vb[15]
        s = _splat(segs_v, bmin)[0]
        e = _splat(segs_v, bmax + 1)[0]

        big = jnp.full((16,), 1e30, jnp.float32)
        zero = jnp.zeros((16,), jnp.int32)
        init6 = (big, big, big, zero, zero, zero)

        def dist(jv):
            dx = px - plsc.load_gather(cx_v, [jv])
            dy = py - plsc.load_gather(cy_v, [jv])
            dz = pz - plsc.load_gather(cz_v, [jv])
            return dx * dx + dy * dy + dz * dz

        def insert(d, jv, acc):
            d1, d2, d3, i1, i2, i3 = acc
            c1 = d < d1
            c2 = d < d2
            c3 = d < d3
            return (jnp.where(c1, d, d1),
                    jnp.where(c1, d1, jnp.where(c2, d, d2)),
                    jnp.where(c2, d2, jnp.where(c3, d, d3)),
                    jnp.where(c1, jv, i1),
                    jnp.where(c1, i1, jnp.where(c2, jv, i2)),
                    jnp.where(c2, i2, jnp.where(c3, jv, i3)))

        def merge_one(d, jv, acc):
            d1, d2, d3, i1, i2, i3 = acc
            c1 = (d < d1) | ((d == d1) & (jv < i1))
            c2 = (d < d2) | ((d == d2) & (jv < i2))
            c3 = (d < d3) | ((d == d3) & (jv < i3))
            return (jnp.where(c1, d, d1),
                    jnp.where(c1, d1, jnp.where(c2, d, d2)),
                    jnp.where(c2, d2, jnp.where(c3, d, d3)),
                    jnp.where(c1, jv, i1),
                    jnp.where(c1, i1, jnp.where(c2, jv, i2)),
                    jnp.where(c2, i2, jnp.where(c3, jv, i3)))

        def fast_scan():
            def quad_body(i, carry):
                a = carry[0:6]
                b = carry[6:12]
                ja = carry[12]
                jb = carry[13]
                a = insert(dist(ja), ja, a)
                b = insert(dist(jb), jb, b)
                ja = ja + 2
                jb = jb + 2
                a = insert(dist(ja), ja, a)
                b = insert(dist(jb), jb, b)
                return a + b + (ja + 2, jb + 2)
            h = (e - s) // 4
            js = jnp.full((16,), s, jnp.int32)
            acc = lax.fori_loop(0, h, quad_body, init6 + init6 + (js, js + 1))

            def rem_body(j, carry):
                return insert(dist(carry[6]), carry[6], carry[0:6]) + (carry[6] + 1,)
            acc_a = lax.fori_loop(s + 4 * h, e, rem_body, acc[0:6] + (acc[12],))
            a = acc_a[0:6]
            b = acc[6:12]
            a = merge_one(b[0], b[3], a)
            a = merge_one(b[1], b[4], a)
            a = merge_one(b[2], b[5], a)
            return a

        def slow_scan():
            def cand_body(j, carry):
                jv = carry[6]
                d = dist(jv)
                d = jnp.where(vb != plsc.load_gather(cb_v, [jv]),
                              jnp.float32(1e10), d)
                return insert(d, jv, carry[0:6]) + (jv + 1,)
            js = jnp.full((16,), s, jnp.int32)
            return lax.fori_loop(s, e, cand_body, init6 + (js,))[0:6]

        d1, d2, d3, i1, i2, i3 = lax.cond(bmin == bmax, fast_scan, slow_scan)

        w1 = 1.0 / jnp.maximum(d1, 1e-16)
        w2 = 1.0 / jnp.maximum(d2, 1e-16)
        w3 = 1.0 / jnp.maximum(d3, 1e-16)
        inv = 1.0 / (w1 + w2 + w3)
        w1_v[pl.ds(o, 16)] = w1 * inv
        w2_v[pl.ds(o, 16)] = w2 * inv
        w3_v[pl.ds(o, 16)] = w3 * inv
        i1_v[pl.ds(o, 16)] = i1
        i2_v[pl.ds(o, 16)] = i2
        i3_v[pl.ds(o, 16)] = i3
        return 0

    lax.fori_loop(0, GPT, group_body, 0)

    for c in range(MPT // CHUNK):
        off = c * CHUNK
        cp1 = pltpu.async_copy(x_h.at[i1_v.at[pl.ds(off, CHUNK)]], r1_v, sem)
        cp2 = pltpu.async_copy(x_h.at[i2_v.at[pl.ds(off, CHUNK)]], r2_v, sem)
        cp3 = pltpu.async_copy(x_h.at[i3_v.at[pl.ds(off, CHUNK)]], r3_v, sem)
        cp1.wait()
        cp2.wait()
        cp3.wait()

        def combine16(g, _):
            m0 = g * 16
            v1 = w1_v[pl.ds(off + m0, 16)]
            v2 = w2_v[pl.ds(off + m0, 16)]
            v3 = w3_v[pl.ds(off + m0, 16)]
            for u in range(16):
                a1 = jnp.full((16,), v1[u], jnp.float32)
                a2 = jnp.full((16,), v2[u], jnp.float32)
                a3 = jnp.full((16,), v3[u], jnp.float32)
                for k in range(D // 16):
                    cs = pl.ds(k * 16, 16)
                    y_v[m0 + u, cs] = (a1 * r1_v[m0 + u, cs]
                                       + a2 * r2_v[m0 + u, cs]
                                       + a3 * r3_v[m0 + u, cs])
            return 0

        lax.fori_loop(0, CHUNK // 16, combine16, 0)
        pltpu.sync_copy(y_v, y_h.at[pl.ds(base + off, CHUNK)])


_knn_call = pl.kernel(
    _knn_body,
    out_type=jax.ShapeDtypeStruct((N_F, D), jnp.float32),
    mesh=plsc.VectorSubcoreMesh(core_axis_name="c", subcore_axis_name="s",
                                num_cores=2, num_subcores=16),
    compiler_params=pltpu.CompilerParams(needs_layout_passes=False,
                                         use_tc_tiling_on_sc=True),
    scratch_types=[
        pltpu.VMEM((N_C,), jnp.float32),
        pltpu.VMEM((N_C,), jnp.float32),
        pltpu.VMEM((N_C,), jnp.float32),
        pltpu.VMEM((N_C,), jnp.int32),
        pltpu.VMEM((MPT,), jnp.float32),
        pltpu.VMEM((MPT,), jnp.float32),
        pltpu.VMEM((MPT,), jnp.float32),
        pltpu.VMEM((MPT,), jnp.int32),
        pltpu.VMEM((16,), jnp.int32),
        pltpu.VMEM((MPT,), jnp.int32),
        pltpu.VMEM((MPT,), jnp.int32),
        pltpu.VMEM((MPT,), jnp.int32),
        pltpu.VMEM((MPT,), jnp.float32),
        pltpu.VMEM((MPT,), jnp.float32),
        pltpu.VMEM((MPT,), jnp.float32),
        pltpu.VMEM((CHUNK, DP), jnp.float32),
        pltpu.VMEM((CHUNK, DP), jnp.float32),
        pltpu.VMEM((CHUNK, DP), jnp.float32),
        pltpu.VMEM((CHUNK, D), jnp.float32),
        pltpu.SemaphoreType.DMA,
    ],
)


def _mlp_body(y_ref, xs_ref, w1_ref, b1_ref, out_ref):
    h = jax.lax.dot(y_ref[...], w1_ref[0:D, :], preferred_element_type=jnp.float32)
    h = h + jax.lax.dot(xs_ref[...], w1_ref[D:, :], preferred_element_type=jnp.float32)
    out_ref[...] = jnp.maximum(h + b1_ref[0, :][None, :], 0.0)


def kernel(x, pos, batch, seed_idx, x_skip, pos_skip, batch_skip, seed_idx_skip, W1, b1):
    pos = pos.astype(jnp.float32)
    ps = pos_skip.astype(jnp.float32)
    bi = batch.astype(jnp.int32)
    fbi = batch_skip.astype(jnp.int32)
    segs = jnp.sum(bi[None, :] < jnp.arange(16, dtype=jnp.int32)[:, None],
                   axis=1).astype(jnp.int32)
    xp = jnp.pad(x.astype(jnp.float32), ((0, 0), (0, DP - D)))

    y = _knn_call(
        pos[:, 0], pos[:, 1], pos[:, 2], bi,
        ps[:, 0], ps[:, 1], ps[:, 2], fbi,
        segs, xp)

    b1r = b1.reshape(1, -1)

    out = pl.pallas_call(
        _mlp_body,
        grid=(N_F // BLK,),
        in_specs=[
            pl.BlockSpec((BLK, D), lambda i: (i, 0)),
            pl.BlockSpec((BLK, D), lambda i: (i, 0)),
            pl.BlockSpec((2 * D, 2 * D), lambda i: (0, 0)),
            pl.BlockSpec((1, 2 * D), lambda i: (0, 0)),
        ],
        out_specs=pl.BlockSpec((BLK, 2 * D), lambda i: (i, 0)),
        out_shape=jax.ShapeDtypeStruct((N_F, 2 * D), jnp.float32),
    )(y, x_skip, W1, b1r)
    return (out, pos_skip, batch_skip)
